# Initial kernel scaffold; baseline (speedup 1.0000x reference)
import functools
import jax
import jax.numpy as jnp
from jax.experimental import pallas as pl
from jax.experimental.pallas import tpu as pltpu

N = 10000
F_IN = 256
HID = 256
HEADS = 4
CLASSES = 40

ROW_BLK = 1000


def _mm1_body(x_ref, w_ref, a_ref, h_ref, logits_ref):
    h = jnp.dot(x_ref[...], w_ref[...], preferred_element_type=jnp.float32)
    h_ref[...] = h
    logits_ref[...] = jnp.dot(h, a_ref[...], preferred_element_type=jnp.float32)


def _proj_with_logits(x, Wt, A, out_dim, n_logit):
    grid = (N // ROW_BLK,)
    return pl.pallas_call(
        _mm1_body,
        grid=grid,
        in_specs=[
            pl.BlockSpec((ROW_BLK, x.shape[1]), lambda i: (i, 0)),
            pl.BlockSpec((x.shape[1], out_dim), lambda i: (0, 0)),
            pl.BlockSpec((out_dim, n_logit), lambda i: (0, 0)),
        ],
        out_specs=[
            pl.BlockSpec((ROW_BLK, out_dim), lambda i: (i, 0)),
            pl.BlockSpec((ROW_BLK, n_logit), lambda i: (i, 0)),
        ],
        out_shape=[
            jax.ShapeDtypeStruct((N, out_dim), jnp.float32),
            jax.ShapeDtypeStruct((N, n_logit), jnp.float32),
        ],
    )(x, Wt, A)


def _edge_phase(h, a_src, a_dst, src, dst, heads, ch):
    alpha = a_src[src] + a_dst[dst]
    alpha = jax.nn.leaky_relu(alpha, negative_slope=0.2)
    ex = jnp.exp(alpha)
    denom = jax.ops.segment_sum(ex, dst, num_segments=N)
    hh = h.reshape(N, heads, ch)
    msg = hh[src] * ex[:, :, None]
    agg = jax.ops.segment_sum(msg, dst, num_segments=N)
    out = agg / (denom[:, :, None] + 1e-16)
    return out.reshape(N, heads * ch)


def kernel(x, edge_index, W1, att_src1, att_dst1, b1, W2, att_src2, att_dst2, b2):
    loop = jnp.arange(N, dtype=edge_index.dtype)
    src = jnp.concatenate([edge_index[0], loop])
    dst = jnp.concatenate([edge_index[1], loop])

    eye = jnp.eye(HEADS, dtype=jnp.float32)
    A1 = jnp.concatenate(
        [
            (att_src1[:, None, :] * eye[:, :, None]).reshape(HEADS * HID, HEADS),
            (att_dst1[:, None, :] * eye[:, :, None]).reshape(HEADS * HID, HEADS),
        ],
        axis=1,
    )

    h1, logits1 = _proj_with_logits(x, W1.T, A1, HEADS * HID, 2 * HEADS)
    out1 = _edge_phase(h1, logits1[:, :HEADS], logits1[:, HEADS:], src, dst, HEADS, HID)
    out1 = jax.nn.relu(out1 + b1)

    A2 = jnp.concatenate([att_src2.T, att_dst2.T], axis=1)
    h2, logits2 = _proj_with_logits(out1, W2.T, A2, CLASSES, 2)
    out2 = _edge_phase(h2, logits2[:, :1], logits2[:, 1:], src, dst, 1, CLASSES)
    out2 = out2 + b2
    return jax.nn.log_softmax(out2, axis=1)


# speedup vs baseline: 1.1273x; 1.1273x over previous
import functools
import jax
import jax.numpy as jnp
from jax.experimental import pallas as pl
from jax.experimental.pallas import tpu as pltpu

N = 10000
F_IN = 256
HID = 256
HEADS = 4
CLASSES = 40

ROW_BLK = 1000


def _mm1_body(x_ref, w_ref, a_ref, h_ref, logits_ref):
    h = jnp.dot(x_ref[...], w_ref[...], preferred_element_type=jnp.float32)
    h_ref[...] = h
    logits_ref[...] = jnp.dot(h, a_ref[...], preferred_element_type=jnp.float32)


def _proj_with_logits(x, Wt, A, out_dim, n_logit):
    grid = (N // ROW_BLK,)
    return pl.pallas_call(
        _mm1_body,
        grid=grid,
        in_specs=[
            pl.BlockSpec((ROW_BLK, x.shape[1]), lambda i: (i, 0)),
            pl.BlockSpec((x.shape[1], out_dim), lambda i: (0, 0)),
            pl.BlockSpec((out_dim, n_logit), lambda i: (0, 0)),
        ],
        out_specs=[
            pl.BlockSpec((ROW_BLK, out_dim), lambda i: (i, 0)),
            pl.BlockSpec((ROW_BLK, n_logit), lambda i: (i, 0)),
        ],
        out_shape=[
            jax.ShapeDtypeStruct((N, out_dim), jnp.float32),
            jax.ShapeDtypeStruct((N, n_logit), jnp.float32),
        ],
    )(x, Wt, A)


def _edge_phase(h, a_src, a_dst, src, dst, heads, ch):
    alpha = a_src[src] + a_dst[dst]
    alpha = jax.nn.leaky_relu(alpha, negative_slope=0.2)
    ex = jnp.exp(alpha)
    denom = jax.ops.segment_sum(ex, dst, num_segments=N)
    hh = h.reshape(N, heads, ch)
    msg = hh[src] * ex[:, :, None]
    agg = jax.ops.segment_sum(msg, dst, num_segments=N)
    out = agg / (denom[:, :, None] + 1e-16)
    return out.reshape(N, heads * ch)


def kernel(x, edge_index, W1, att_src1, att_dst1, b1, W2, att_src2, att_dst2, b2):
    loop = jnp.arange(N, dtype=edge_index.dtype)
    src = jnp.concatenate([edge_index[0], loop])
    dst = jnp.concatenate([edge_index[1], loop])

    eye = jnp.eye(HEADS, dtype=jnp.float32)
    A1 = jnp.concatenate(
        [
            (att_src1[:, :, None] * eye[:, None, :]).reshape(HEADS * HID, HEADS),
            (att_dst1[:, :, None] * eye[:, None, :]).reshape(HEADS * HID, HEADS),
        ],
        axis=1,
    )

    h1, logits1 = _proj_with_logits(x, W1.T, A1, HEADS * HID, 2 * HEADS)
    out1 = _edge_phase(h1, logits1[:, :HEADS], logits1[:, HEADS:], src, dst, HEADS, HID)
    out1 = jax.nn.relu(out1 + b1)

    A2 = jnp.concatenate([att_src2.T, att_dst2.T], axis=1)
    h2, logits2 = _proj_with_logits(out1, W2.T, A2, CLASSES, 2)
    out2 = _edge_phase(h2, logits2[:, :1], logits2[:, 1:], src, dst, 1, CLASSES)
    out2 = out2 + b2
    return jax.nn.log_softmax(out2, axis=1)
